# baseline (speedup 1.0000x reference)
import functools

import jax
import jax.numpy as jnp
from jax.experimental import pallas as pl
from jax.experimental.pallas import tpu as pltpu


def _support_body(f_ref, w_ref, out_ref):
    out_ref[...] = jnp.maximum(
        jnp.dot(f_ref[...], w_ref[...], preferred_element_type=jnp.float32), 0.0
    )


def _pass1_body(adj_ref, x_ref, s_ref, t_ref, low_ref):
    t = jnp.dot(adj_ref[...], x_ref[...], preferred_element_type=jnp.float32)
    t_ref[...] = t
    low_ref[...] = t + s_ref[...]


def _pass2_body(adj_ref, x_ref, s_ref, low_ref, mid_ref, cat_ref):
    t = jnp.dot(adj_ref[...], x_ref[...], preferred_element_type=jnp.float32)
    mid = t - s_ref[...]
    mid_ref[...] = mid
    cat_ref[:, 0, :] = low_ref[...]
    cat_ref[:, 1, :] = mid


@functools.partial(jax.jit, static_argnames=())
def kernel(feature, adj, W):
    n, f_in = feature.shape
    d = W.shape[1]
    dtype = feature.dtype

    bm_s = 512
    support = pl.pallas_call(
        _support_body,
        grid=(n // bm_s,),
        in_specs=[
            pl.BlockSpec((bm_s, f_in), lambda i: (i, 0)),
            pl.BlockSpec((f_in, d), lambda i: (0, 0)),
        ],
        out_specs=pl.BlockSpec((bm_s, d), lambda i: (i, 0)),
        out_shape=jax.ShapeDtypeStruct((n, d), dtype),
        compiler_params=pltpu.CompilerParams(
            dimension_semantics=("arbitrary",)
        ),
    )(feature, W)

    bm = 1024
    grid = (n // bm,)
    adj_spec = pl.BlockSpec((bm, n), lambda i: (i, 0))
    full_spec = pl.BlockSpec((n, d), lambda i: (0, 0))
    row_spec = pl.BlockSpec((bm, d), lambda i: (i, 0))
    row_shape = jax.ShapeDtypeStruct((n, d), dtype)
    params = pltpu.CompilerParams(dimension_semantics=("arbitrary",))

    t1, out_low = pl.pallas_call(
        _pass1_body,
        grid=grid,
        in_specs=[adj_spec, full_spec, row_spec],
        out_specs=[row_spec, row_spec],
        out_shape=[row_shape, row_shape],
        compiler_params=params,
    )(adj, support, support)

    out_mid, output = pl.pallas_call(
        _pass2_body,
        grid=grid,
        in_specs=[adj_spec, full_spec, row_spec, row_spec],
        out_specs=[row_spec, pl.BlockSpec((bm, 2, d), lambda i: (i, 0, 0))],
        out_shape=[row_shape, jax.ShapeDtypeStruct((n, 2, d), dtype)],
        compiler_params=params,
    )(adj, t1, support, out_low)

    return (output, out_low, out_mid)


# speedup vs baseline: 1.0751x; 1.0751x over previous
import functools

import jax
import jax.numpy as jnp
from jax.experimental import pallas as pl
from jax.experimental.pallas import tpu as pltpu


def _support_body(f_ref, w_ref, out_ref):
    out_ref[...] = jnp.maximum(
        jnp.dot(f_ref[...], w_ref[...], preferred_element_type=jnp.float32), 0.0
    )


def _pass1_body(adj_ref, x_ref, s_ref, t_ref, low_ref):
    t = jnp.dot(adj_ref[...], x_ref[...], preferred_element_type=jnp.float32)
    t_ref[...] = t
    low_ref[...] = t + s_ref[...]


def _pass2_body(adj_ref, x_ref, s_ref, low_ref, mid_ref, cat_ref):
    t = jnp.dot(adj_ref[...], x_ref[...], preferred_element_type=jnp.float32)
    mid = t - s_ref[...]
    mid_ref[...] = mid
    cat_ref[:, 0, :] = low_ref[...]
    cat_ref[:, 1, :] = mid


@functools.partial(jax.jit, static_argnames=())
def kernel(feature, adj, W):
    n, f_in = feature.shape
    d = W.shape[1]
    dtype = feature.dtype

    bm_s = 512
    support = pl.pallas_call(
        _support_body,
        grid=(n // bm_s,),
        in_specs=[
            pl.BlockSpec((bm_s, f_in), lambda i: (i, 0)),
            pl.BlockSpec((f_in, d), lambda i: (0, 0)),
        ],
        out_specs=pl.BlockSpec((bm_s, d), lambda i: (i, 0)),
        out_shape=jax.ShapeDtypeStruct((n, d), dtype),
        compiler_params=pltpu.CompilerParams(
            dimension_semantics=("parallel",)
        ),
    )(feature, W)

    bm = 512
    grid = (n // bm,)
    adj_spec = pl.BlockSpec((bm, n), lambda i: (i, 0))
    full_spec = pl.BlockSpec((n, d), lambda i: (0, 0))
    row_spec = pl.BlockSpec((bm, d), lambda i: (i, 0))
    row_shape = jax.ShapeDtypeStruct((n, d), dtype)
    params = pltpu.CompilerParams(dimension_semantics=("parallel",))

    t1, out_low = pl.pallas_call(
        _pass1_body,
        grid=grid,
        in_specs=[adj_spec, full_spec, row_spec],
        out_specs=[row_spec, row_spec],
        out_shape=[row_shape, row_shape],
        compiler_params=params,
    )(adj, support, support)

    out_mid, output = pl.pallas_call(
        _pass2_body,
        grid=grid,
        in_specs=[adj_spec, full_spec, row_spec, row_spec],
        out_specs=[row_spec, pl.BlockSpec((bm, 2, d), lambda i: (i, 0, 0))],
        out_shape=[row_shape, jax.ShapeDtypeStruct((n, 2, d), dtype)],
        compiler_params=params,
    )(adj, t1, support, out_low)

    return (output, out_low, out_mid)


# speedup vs baseline: 1.3656x; 1.2702x over previous
import functools

import jax
import jax.numpy as jnp
from jax.experimental import pallas as pl
from jax.experimental.pallas import tpu as pltpu


def _body(f_ref, w_ref, adj_s_ref, adj_any_ref, low_ref, mid_ref, cat_ref,
          sup_s, t1_s, res_s, sem, *, bm, nblk, nres):
    g = pl.program_id(0)
    half = nres * bm

    @pl.when(g == 0)
    def _():
        pltpu.make_async_copy(
            adj_any_ref.at[pl.ds(0, half), :], res_s, sem
        ).start()
        sup_s[...] = jnp.maximum(
            jnp.dot(f_ref[...], w_ref[...], preferred_element_type=jnp.float32), 0.0
        )

    nstream = nblk - nres

    @pl.when((g >= 1) & (g <= nstream))
    def _():
        r = (nres + g - 1) * bm
        t = jnp.dot(adj_s_ref[...], sup_s[...], preferred_element_type=jnp.float32)
        t1_s[pl.ds(r, bm), :] = t
        low_ref[...] = t + sup_s[pl.ds(r, bm), :]

    @pl.when(g == nstream + 1)
    def _():
        pltpu.make_async_copy(
            adj_any_ref.at[pl.ds(0, half), :], res_s, sem
        ).wait()

    @pl.when((g >= nstream + 1) & (g <= nblk))
    def _():
        b = g - nstream - 1
        r = b * bm
        t = jnp.dot(res_s[pl.ds(r, bm), :], sup_s[...],
                    preferred_element_type=jnp.float32)
        t1_s[pl.ds(r, bm), :] = t
        low_ref[...] = t + sup_s[pl.ds(r, bm), :]

    @pl.when((g >= nblk + 1) & (g <= nblk + nstream))
    def _():
        r = (2 * nblk - g) * bm
        t2 = jnp.dot(adj_s_ref[...], t1_s[...], preferred_element_type=jnp.float32)
        mid = t2 - sup_s[pl.ds(r, bm), :]
        mid_ref[...] = mid
        cat_ref[:, 0, :] = t1_s[pl.ds(r, bm), :] + sup_s[pl.ds(r, bm), :]
        cat_ref[:, 1, :] = mid

    @pl.when(g > nblk + nstream)
    def _():
        b = g - nblk - nstream - 1
        r = b * bm
        t2 = jnp.dot(res_s[pl.ds(r, bm), :], t1_s[...],
                     preferred_element_type=jnp.float32)
        mid = t2 - sup_s[pl.ds(r, bm), :]
        mid_ref[...] = mid
        cat_ref[:, 0, :] = t1_s[pl.ds(r, bm), :] + sup_s[pl.ds(r, bm), :]
        cat_ref[:, 1, :] = mid


@jax.jit
def kernel(feature, adj, W):
    n, f_in = feature.shape
    d = W.shape[1]
    dtype = feature.dtype

    bm = 512
    nblk = n // bm
    nres = nblk // 2
    nstream = nblk - nres

    def stream_idx(g):
        p1 = jnp.clip(nres + g - 1, nres, nblk - 1)
        p2 = jnp.clip(2 * nblk - g, nres, nblk - 1)
        return (jnp.where(g <= nblk, p1, p2), 0)

    def row_of(g):
        gg = jnp.clip(g, 1, nblk)
        b1 = jnp.where(gg <= nstream, nres + gg - 1, gg - nstream - 1)
        return b1

    def row_of2(g):
        gg = jnp.clip(g, nblk + 1, 2 * nblk)
        b2 = jnp.where(gg <= nblk + nstream, 2 * nblk - gg, gg - nblk - nstream - 1)
        return b2

    out_low, out_mid, output = pl.pallas_call(
        functools.partial(_body, bm=bm, nblk=nblk, nres=nres),
        grid=(2 * nblk + 1,),
        in_specs=[
            pl.BlockSpec((n, f_in), lambda g: (0, 0)),
            pl.BlockSpec((f_in, d), lambda g: (0, 0)),
            pl.BlockSpec((bm, n), stream_idx),
            pl.BlockSpec(memory_space=pl.ANY),
        ],
        out_specs=[
            pl.BlockSpec((bm, d), lambda g: (row_of(g), 0)),
            pl.BlockSpec((bm, d), lambda g: (row_of2(g), 0)),
            pl.BlockSpec((bm, 2, d), lambda g: (row_of2(g), 0, 0)),
        ],
        out_shape=[
            jax.ShapeDtypeStruct((n, d), dtype),
            jax.ShapeDtypeStruct((n, d), dtype),
            jax.ShapeDtypeStruct((n, 2, d), dtype),
        ],
        scratch_shapes=[
            pltpu.VMEM((n, d), jnp.float32),
            pltpu.VMEM((n, d), jnp.float32),
            pltpu.VMEM((nres * bm, n), jnp.float32),
            pltpu.SemaphoreType.DMA,
        ],
        compiler_params=pltpu.CompilerParams(
            dimension_semantics=("arbitrary",)
        ),
    )(feature, W, adj, adj)

    return (output, out_low, out_mid)


# speedup vs baseline: 1.4124x; 1.0343x over previous
import functools

import jax
import jax.numpy as jnp
from jax.experimental import pallas as pl
from jax.experimental.pallas import tpu as pltpu


def _body(f_ref, w_ref, adj_s_ref, adj_any_ref, low_ref, mid_ref, cat_ref,
          sup_s, t1_s, res_s, sem, *, bm, nblk, nres):
    g = pl.program_id(0)
    half = nres * bm

    @pl.when(g < nres)
    def _():
        pltpu.make_async_copy(
            adj_any_ref.at[pl.ds(g * bm, bm), :],
            res_s.at[pl.ds(g * bm, bm), :],
            sem,
        ).start()

    @pl.when(g == 0)
    def _():
        sup_s[...] = jnp.maximum(
            jnp.dot(f_ref[...], w_ref[...], preferred_element_type=jnp.float32), 0.0
        )

    nstream = nblk - nres

    @pl.when((g >= 1) & (g <= nstream))
    def _():
        r = (nres + g - 1) * bm
        t = jnp.dot(adj_s_ref[...], sup_s[...], preferred_element_type=jnp.float32)
        t1_s[pl.ds(r, bm), :] = t
        low_ref[...] = t + sup_s[pl.ds(r, bm), :]

    @pl.when(g == nstream + 1)
    def _():
        for _ in range(nres):
            pltpu.make_async_copy(
                adj_any_ref.at[pl.ds(0, bm), :],
                res_s.at[pl.ds(0, bm), :],
                sem,
            ).wait()

    @pl.when((g >= nstream + 1) & (g <= nblk))
    def _():
        b = g - nstream - 1
        r = b * bm
        t = jnp.dot(res_s[pl.ds(r, bm), :], sup_s[...],
                    preferred_element_type=jnp.float32)
        t1_s[pl.ds(r, bm), :] = t
        low_ref[...] = t + sup_s[pl.ds(r, bm), :]

    @pl.when((g >= nblk + 1) & (g <= nblk + nstream))
    def _():
        r = (2 * nblk - g) * bm
        t2 = jnp.dot(adj_s_ref[...], t1_s[...], preferred_element_type=jnp.float32)
        mid = t2 - sup_s[pl.ds(r, bm), :]
        mid_ref[...] = mid
        cat_ref[:, 0, :] = t1_s[pl.ds(r, bm), :] + sup_s[pl.ds(r, bm), :]
        cat_ref[:, 1, :] = mid

    @pl.when(g > nblk + nstream)
    def _():
        b = g - nblk - nstream - 1
        r = b * bm
        t2 = jnp.dot(res_s[pl.ds(r, bm), :], t1_s[...],
                     preferred_element_type=jnp.float32)
        mid = t2 - sup_s[pl.ds(r, bm), :]
        mid_ref[...] = mid
        cat_ref[:, 0, :] = t1_s[pl.ds(r, bm), :] + sup_s[pl.ds(r, bm), :]
        cat_ref[:, 1, :] = mid


@jax.jit
def kernel(feature, adj, W):
    n, f_in = feature.shape
    d = W.shape[1]
    dtype = feature.dtype

    bm = 512
    nblk = n // bm
    nres = nblk // 2
    nstream = nblk - nres

    def stream_idx(g):
        p1 = jnp.clip(nres + g - 1, nres, nblk - 1)
        p2 = jnp.clip(2 * nblk - g, nres, nblk - 1)
        return (jnp.where(g <= nblk, p1, p2), 0)

    def row_of(g):
        gg = jnp.clip(g, 1, nblk)
        b1 = jnp.where(gg <= nstream, nres + gg - 1, gg - nstream - 1)
        return b1

    def row_of2(g):
        gg = jnp.clip(g, nblk + 1, 2 * nblk)
        b2 = jnp.where(gg <= nblk + nstream, 2 * nblk - gg, gg - nblk - nstream - 1)
        return b2

    out_low, out_mid, output = pl.pallas_call(
        functools.partial(_body, bm=bm, nblk=nblk, nres=nres),
        grid=(2 * nblk + 1,),
        in_specs=[
            pl.BlockSpec((n, f_in), lambda g: (0, 0)),
            pl.BlockSpec((f_in, d), lambda g: (0, 0)),
            pl.BlockSpec((bm, n), stream_idx),
            pl.BlockSpec(memory_space=pl.ANY),
        ],
        out_specs=[
            pl.BlockSpec((bm, d), lambda g: (row_of(g), 0)),
            pl.BlockSpec((bm, d), lambda g: (row_of2(g), 0)),
            pl.BlockSpec((bm, 2, d), lambda g: (row_of2(g), 0, 0)),
        ],
        out_shape=[
            jax.ShapeDtypeStruct((n, d), dtype),
            jax.ShapeDtypeStruct((n, d), dtype),
            jax.ShapeDtypeStruct((n, 2, d), dtype),
        ],
        scratch_shapes=[
            pltpu.VMEM((n, d), jnp.float32),
            pltpu.VMEM((n, d), jnp.float32),
            pltpu.VMEM((nres * bm, n), jnp.float32),
            pltpu.SemaphoreType.DMA,
        ],
        compiler_params=pltpu.CompilerParams(
            dimension_semantics=("arbitrary",)
        ),
    )(feature, W, adj, adj)

    return (output, out_low, out_mid)
